# baseline (speedup 1.0000x reference)
import functools

import jax
import jax.numpy as jnp
from jax import lax
from jax.experimental import pallas as pl
from jax.experimental.pallas import tpu as pltpu
from jax.experimental.pallas import tpu_sc as plsc

N_BANK = 1_000_000
D_DIM = 128
B_TOK = 16384

_NC = 2
_NS = 16
_NW = _NC * _NS
_ROWS_PER_W = D_DIM // _NW

_mesh = plsc.VectorSubcoreMesh(core_axis_name="c", subcore_axis_name="s")


@functools.partial(
    pl.kernel,
    mesh=_mesh,
    out_type=jax.ShapeDtypeStruct((D_DIM, B_TOK), jnp.float32),
    scratch_types=[
        pltpu.VMEM((B_TOK,), jnp.int32),
        pltpu.VMEM((B_TOK,), jnp.float32),
        pltpu.SemaphoreType.DMA,
    ],
)
def _gather_rows(idx_hbm, mb_hbm, out_hbm, idx_v, row_v, sem):
    wid = lax.axis_index("s") * _NC + lax.axis_index("c")
    pltpu.sync_copy(idx_hbm, idx_v)
    for r in range(_ROWS_PER_W):
        d = wid * _ROWS_PER_W + r
        row_view = mb_hbm.at[pl.ds(d * N_BANK, N_BANK)]
        pltpu.async_copy(row_view.at[idx_v], row_v, sem).wait()
        pltpu.sync_copy(row_v, out_hbm.at[d])


def kernel(index, membank):
    mb_flat = membank.reshape(D_DIM * N_BANK)
    return _gather_rows(index, mb_flat)


# speedup vs baseline: 291.3769x; 291.3769x over previous
import functools

import jax
import jax.numpy as jnp
from jax import lax
from jax.experimental import pallas as pl
from jax.experimental.pallas import tpu as pltpu
from jax.experimental.pallas import tpu_sc as plsc

N_BANK = 1_000_000
D_DIM = 128
B_TOK = 16384

_NC = 2
_NS = 16
_NW = _NC * _NS
_B_PER_W = B_TOK // _NW

_mesh = plsc.VectorSubcoreMesh(core_axis_name="c", subcore_axis_name="s")


@functools.partial(
    pl.kernel,
    mesh=_mesh,
    out_type=jax.ShapeDtypeStruct((B_TOK, D_DIM), jnp.float32),
    scratch_types=[
        pltpu.VMEM((_B_PER_W,), jnp.int32),
        pltpu.VMEM((_B_PER_W, D_DIM), jnp.float32),
        pltpu.SemaphoreType.DMA,
    ],
)
def _gather_rows(idx_hbm, mbt_hbm, out_hbm, idx_v, rows_v, sem):
    wid = lax.axis_index("s") * _NC + lax.axis_index("c")
    base = wid * _B_PER_W
    pltpu.sync_copy(idx_hbm.at[pl.ds(base, _B_PER_W)], idx_v)
    pltpu.async_copy(mbt_hbm.at[idx_v], rows_v, sem).wait()
    pltpu.sync_copy(rows_v, out_hbm.at[pl.ds(base, _B_PER_W)])


def kernel(index, membank):
    mbt = membank.T
    out_t = _gather_rows(index, mbt)
    return out_t.T
